# baseline (speedup 1.0000x reference)
import functools

import jax
import jax.numpy as jnp
from jax.experimental import pallas as pl
from jax.experimental.pallas import tpu as pltpu

_KS = 5
_K = 125


def _conv_mm_kernel(a_ref, x_ref, wf_ref, root_ref, b_ref, o_ref):
    acc = jnp.dot(a_ref[...], wf_ref[...], preferred_element_type=jnp.float32)
    acc = acc + jnp.dot(x_ref[...], root_ref[...],
                        preferred_element_type=jnp.float32)
    acc = acc + b_ref[...]
    o_ref[...] = jnp.where(acc > 0.0, acc, jnp.exp(acc) - 1.0)


def _conv_matmul(a2, x, w, root, b):
    n, kcin = a2.shape
    cin = x.shape[1]
    cout = w.shape[2]
    wf = w.reshape(_K * cin, cout)
    bn = 400
    grid = (n // bn,)
    return pl.pallas_call(
        _conv_mm_kernel,
        grid=grid,
        in_specs=[
            pl.BlockSpec((bn, kcin), lambda i: (i, 0)),
            pl.BlockSpec((bn, cin), lambda i: (i, 0)),
            pl.BlockSpec((kcin, cout), lambda i: (0, 0)),
            pl.BlockSpec((cin, cout), lambda i: (0, 0)),
            pl.BlockSpec((1, cout), lambda i: (0, 0)),
        ],
        out_specs=pl.BlockSpec((bn, cout), lambda i: (i, 0)),
        out_shape=jax.ShapeDtypeStruct((n, cout), jnp.float32),
    )(a2, x, wf, root, b.reshape(1, cout))


def _head_kernel(h_ref, w1_ref, b1_ref, w2_ref, b2_ref, o_ref):
    h1 = jnp.dot(h_ref[...], w1_ref[...], preferred_element_type=jnp.float32)
    h1 = h1 + b1_ref[...]
    h1 = jnp.where(h1 > 0.0, h1, jnp.exp(h1) - 1.0)
    z = jnp.dot(h1, w2_ref[...], preferred_element_type=jnp.float32)
    z = z + b2_ref[...]
    m = jnp.max(z, axis=1, keepdims=True)
    zs = z - m
    lse = jnp.log(jnp.sum(jnp.exp(zs), axis=1, keepdims=True))
    o_ref[...] = zs - lse


def _fc_head(h, fc1w, fc1b, fc2w, fc2b):
    n, c = h.shape
    cmid = fc1w.shape[1]
    nout = fc2w.shape[1]
    npad = ((nout + 127) // 128) * 128
    w2 = jnp.zeros((cmid, npad), jnp.float32).at[:, :nout].set(fc2w)
    b2 = jnp.full((1, npad), -1e30, jnp.float32).at[0, :nout].set(fc2b)
    bn = 400
    out = pl.pallas_call(
        _head_kernel,
        grid=(n // bn,),
        in_specs=[
            pl.BlockSpec((bn, c), lambda i: (i, 0)),
            pl.BlockSpec((c, cmid), lambda i: (0, 0)),
            pl.BlockSpec((1, cmid), lambda i: (0, 0)),
            pl.BlockSpec((cmid, npad), lambda i: (0, 0)),
            pl.BlockSpec((1, npad), lambda i: (0, 0)),
        ],
        out_specs=pl.BlockSpec((bn, npad), lambda i: (i, 0)),
        out_shape=jax.ShapeDtypeStruct((n, npad), jnp.float32),
    )(h, fc1w, fc1b.reshape(1, cmid), w2, b2)
    return out[:, :nout]


def kernel(x, edge_index, pseudo, W1, root1, b1, W2, root2, b2, W3, root3,
           b3, W4, root4, b4, W5, root5, b5, W6, root6, b6, fc1W, fc1b,
           fc2W, fc2b):
    n = x.shape[0]
    row, col = edge_index[0], edge_index[1]

    v = pseudo * (_KS - 1.0)
    i0 = jnp.floor(v).astype(jnp.int32)
    f = v - jnp.floor(v)
    deg = jnp.zeros((n,), jnp.float32).at[col].add(1.0)
    inv_deg = 1.0 / jnp.clip(deg, 1.0, None)
    scale = inv_deg[col]

    bas_list = []
    t_list = []
    for s0 in (0, 1):
        for s1 in (0, 1):
            for s2 in (0, 1):
                b0 = f[:, 0] if s0 else (1.0 - f[:, 0])
                b1_ = f[:, 1] if s1 else (1.0 - f[:, 1])
                b2_ = f[:, 2] if s2 else (1.0 - f[:, 2])
                bas_list.append(b0 * b1_ * b2_ * scale)
                wi = (i0[:, 0] + s0) + (i0[:, 1] + s1) * _KS \
                    + (i0[:, 2] + s2) * (_KS * _KS)
                t_list.append(col * _K + wi)
    bas = jnp.stack(bas_list)
    tgt = jnp.stack(t_list)

    h = x
    for (W, r, b) in ((W1, root1, b1), (W2, root2, b2), (W3, root3, b3),
                      (W4, root4, b4), (W5, root5, b5), (W6, root6, b6)):
        cin = h.shape[1]
        hsrc = h[row]
        a2 = jnp.zeros((n * _K, cin), jnp.float32)
        for c in range(8):
            a2 = a2.at[tgt[c]].add(bas[c][:, None] * hsrc)
        h = _conv_matmul(a2.reshape(n, _K * cin), h, W, r, b)

    return _fc_head(h, fc1W, fc1b, fc2W, fc2b)
